# Initial kernel scaffold; baseline (speedup 1.0000x reference)
import jax
import jax.numpy as jnp
from jax.experimental import pallas as pl


def kernel(x, edge_index, batch, W_enc, b_enc, W_r1, b_r1, W_r2, b_r2, We0_root, We0_nbr, be0, We1_root, We1_nbr, be1):
    raise NotImplementedError("write your pallas kernel here")


# speedup vs baseline: 11.1039x; 11.1039x over previous
import functools

import jax
import jax.numpy as jnp
from jax import lax
from jax.experimental import pallas as pl
from jax.experimental.pallas import tpu as pltpu, tpu_sc as plsc

N = 10000
E = 160000
IN_DIM = 256
HIDDEN = 512
OUT_DIM = 256
NEXP = 8
NGRAPH = 64

C = 4
CW = 128
BN = 400
NBLK = N // BN
NPAD = 10240
EPAD = 163840
NSUB = 16
EPS = EPAD // NSUB
EB = 128
NEB = EPS // EB
TRASH = N
ACC_ROWS = N + 16

HI = jax.lax.Precision.HIGHEST
DEF = jax.lax.Precision.DEFAULT


def _enc_body(x_ref, w_ref, b_ref, h_ref, ht_ref):
    h = jnp.maximum(jnp.dot(x_ref[...], w_ref[...], precision=DEF) + b_ref[...], 0.0)
    h_ref[...] = h
    for c in range(C):
        ht_ref[c] = h[:, c * CW:(c + 1) * CW]


def _encoder(x, W_enc, b_enc):
    return pl.pallas_call(
        _enc_body,
        grid=(NBLK,),
        in_specs=[
            pl.BlockSpec((BN, IN_DIM), lambda i: (i, 0)),
            pl.BlockSpec((IN_DIM, HIDDEN), lambda i: (0, 0)),
            pl.BlockSpec((1, HIDDEN), lambda i: (0, 0)),
        ],
        out_specs=[
            pl.BlockSpec((BN, HIDDEN), lambda i: (i, 0)),
            pl.BlockSpec((C, BN, CW), lambda i: (0, i, 0)),
        ],
        out_shape=[
            jax.ShapeDtypeStruct((N, HIDDEN), jnp.float32),
            jax.ShapeDtypeStruct((C, NPAD, CW), jnp.float32),
        ],
    )(x, W_enc, b_enc.reshape(1, HIDDEN))


def _sc_agg0_body(ht, srcp, dstp, zbig,
                  agg0,
                  src_v, dst_v, rows_v, sem,
                  acc):
    core = lax.axis_index("c")
    sid = lax.axis_index("s")
    ebase = sid * EPS

    for j in range(2):
        chunk = core * 2 + j

        @pl.when(sid == 0)
        def _init():
            pltpu.sync_copy(zbig, acc)

        plsc.subcore_barrier()

        def _edge_block(b, carry):
            off = ebase + b * EB
            pltpu.sync_copy(srcp.at[pl.ds(off, EB)], src_v)
            pltpu.sync_copy(dstp.at[pl.ds(off, EB)], dst_v)
            pltpu.async_copy(ht.at[chunk].at[src_v], rows_v, sem).wait()
            pltpu.sync_copy(rows_v, acc.at[dst_v], add=True)
            return carry

        lax.fori_loop(0, NEB, _edge_block, 0)
        plsc.subcore_barrier()

        @pl.when(sid == 0)
        def _writeback():
            pltpu.sync_copy(acc.at[pl.ds(0, N)], agg0.at[chunk])


def _sc_agg0(ht, srcp, dstp, zbig):
    mesh = plsc.VectorSubcoreMesh(core_axis_name="c", subcore_axis_name="s")
    fn = pl.kernel(
        _sc_agg0_body, mesh=mesh,
        out_type=jax.ShapeDtypeStruct((C, N, CW), jnp.float32),
        scratch_types=[
            pltpu.VMEM((EB,), jnp.int32),
            pltpu.VMEM((EB,), jnp.int32),
            pltpu.VMEM((EB, CW), jnp.float32),
            pltpu.SemaphoreType.DMA,
            pltpu.VMEM_SHARED((ACC_ROWS, CW), jnp.float32),
        ],
    )
    return fn(ht, srcp, dstp, zbig)


def _counts_body(b_ref, s_ref, nc_ref, ec_ref):
    b = b_ref[...]
    srcv = s_ref[...]
    nc = []
    cnt = []
    for g in range(NGRAPH + 1):
        if g < NGRAPH:
            nc.append(jnp.sum((b == g).astype(jnp.float32)))
        start_g = jnp.sum((b < g).astype(jnp.int32))
        cnt.append(jnp.sum((srcv >= start_g).astype(jnp.float32)))
    ncv = jnp.stack(nc)
    cntv = jnp.stack(cnt)
    ecv = cntv[:-1] - cntv[1:]
    nc_ref[...] = jnp.broadcast_to(ncv[:, None], (NGRAPH, 16))
    ec_ref[...] = jnp.broadcast_to(ecv[:, None], (NGRAPH, 16))


def _counts(batchp2, src2):
    return pl.pallas_call(
        _counts_body,
        grid=(1,),
        in_specs=[
            pl.BlockSpec((NPAD // 128, 128), lambda i: (0, 0)),
            pl.BlockSpec((E // 128, 128), lambda i: (0, 0)),
        ],
        out_specs=[
            pl.BlockSpec((NGRAPH, 16), lambda i: (0, 0)),
            pl.BlockSpec((NGRAPH, 16), lambda i: (0, 0)),
        ],
        out_shape=[
            jax.ShapeDtypeStruct((NGRAPH, 16), jnp.float32),
            jax.ShapeDtypeStruct((NGRAPH, 16), jnp.float32),
        ],
    )(batchp2, src2)


def _router_body(h_ref, b3_ref, nc_ref, ec_ref, w1a_ref, w1b_ref, b1_ref,
                 w2_ref, b2_ref, idx_ref, hist_ref):
    i = pl.program_id(0)
    b = b3_ref[0, 0, :]
    counts = jnp.concatenate([nc_ref[:, 0:1], ec_ref[:, 0:1]], axis=1)
    oh = (b[:, None] == lax.broadcasted_iota(jnp.int32, (1, NGRAPH), 1)
          ).astype(jnp.float32)
    sf = jnp.log1p(jnp.dot(oh, counts, precision=HI))
    r = jnp.maximum(
        jnp.dot(h_ref[...], w1a_ref[...], precision=DEF)
        + jnp.dot(sf, w1b_ref[...], precision=DEF) + b1_ref[...], 0.0)
    logits = jnp.dot(r, w2_ref[...], precision=DEF) + b2_ref[...]
    idx = jnp.argmax(logits, axis=-1).astype(jnp.int32)
    idx_ref[0, 0, :] = idx
    hist = jnp.sum(
        (idx[:, None] == lax.broadcasted_iota(jnp.int32, (1, NEXP), 1)
         ).astype(jnp.int32), axis=0)[None, :]

    @pl.when(i == 0)
    def _first():
        hist_ref[...] = hist

    @pl.when(i > 0)
    def _rest():
        hist_ref[...] = hist_ref[...] + hist


def _router(h, batch3, nc, ec, W_r1, b_r1, W_r2, b_r2):
    return pl.pallas_call(
        _router_body,
        grid=(NBLK,),
        in_specs=[
            pl.BlockSpec((BN, HIDDEN), lambda i: (i, 0)),
            pl.BlockSpec((1, 1, BN), lambda i: (i, 0, 0)),
            pl.BlockSpec((NGRAPH, 16), lambda i: (0, 0)),
            pl.BlockSpec((NGRAPH, 16), lambda i: (0, 0)),
            pl.BlockSpec((HIDDEN, HIDDEN), lambda i: (0, 0)),
            pl.BlockSpec((2, HIDDEN), lambda i: (0, 0)),
            pl.BlockSpec((1, HIDDEN), lambda i: (0, 0)),
            pl.BlockSpec((HIDDEN, NEXP), lambda i: (0, 0)),
            pl.BlockSpec((1, NEXP), lambda i: (0, 0)),
        ],
        out_specs=[
            pl.BlockSpec((1, 1, BN), lambda i: (i, 0, 0)),
            pl.BlockSpec((1, NEXP), lambda i: (0, 0)),
        ],
        out_shape=[
            jax.ShapeDtypeStruct((NBLK, 1, BN), jnp.int32),
            jax.ShapeDtypeStruct((1, NEXP), jnp.int32),
        ],
    )(h, batch3, nc, ec, W_r1[:HIDDEN], W_r1[HIDDEN:], b_r1.reshape(1, HIDDEN),
      W_r2, b_r2.reshape(1, NEXP))


def _l0_body(h_ref, a_ref, wr_ref, wn_ref, b_ref, out_ref):
    agg = jnp.concatenate([a_ref[c] for c in range(C)], axis=1)
    z = jnp.maximum(
        jnp.dot(h_ref[...], wr_ref[0], precision=DEF)
        + jnp.dot(agg, wn_ref[0], precision=DEF) + b_ref[0], 0.0)
    for c in range(C):
        out_ref[0, c] = z[:, c * CW:(c + 1) * CW]


def _layer0(h, agg0, We0_root, We0_nbr, be0):
    return pl.pallas_call(
        _l0_body,
        grid=(NEXP, NBLK),
        in_specs=[
            pl.BlockSpec((BN, HIDDEN), lambda e, i: (i, 0)),
            pl.BlockSpec((C, BN, CW), lambda e, i: (0, i, 0)),
            pl.BlockSpec((1, HIDDEN, HIDDEN), lambda e, i: (e, 0, 0)),
            pl.BlockSpec((1, HIDDEN, HIDDEN), lambda e, i: (e, 0, 0)),
            pl.BlockSpec((1, 1, HIDDEN), lambda e, i: (e, 0, 0)),
        ],
        out_specs=pl.BlockSpec((1, C, BN, CW), lambda e, i: (e, 0, i, 0)),
        out_shape=jax.ShapeDtypeStruct((NEXP, C, NPAD, CW), jnp.float32),
    )(h, agg0, We0_root, We0_nbr, be0.reshape(NEXP, 1, HIDDEN))


def _sc_agg1_body(h1f, srcp, dstp, idxp, zbig,
                  agg1, h1sel,
                  ev_v, src_v, dst_v, comb_v, rows_v, sem,
                  acc):
    core = lax.axis_index("c")
    sid = lax.axis_index("s")
    ebase = sid * EPS

    for j in range(2):
        chunk = core * 2 + j

        @pl.when(sid == 0)
        def _init():
            pltpu.sync_copy(zbig, acc)

        plsc.subcore_barrier()

        def _edge_block(b, carry):
            off = ebase + b * EB
            pltpu.sync_copy(srcp.at[pl.ds(off, EB)], src_v)
            pltpu.sync_copy(dstp.at[pl.ds(off, EB)], dst_v)
            pltpu.async_copy(idxp.at[dst_v], ev_v, sem).wait()
            for k in range(8):
                kk = pl.ds(k * 16, 16)
                comb_v[kk] = ev_v[kk] * (C * NPAD) + chunk * NPAD + src_v[kk]
            pltpu.async_copy(h1f.at[comb_v], rows_v, sem).wait()
            pltpu.sync_copy(rows_v, acc.at[dst_v], add=True)
            return carry

        lax.fori_loop(0, NEB, _edge_block, 0)

        for t in range(5):
            n0 = sid * 640 + t * EB
            pltpu.sync_copy(idxp.at[pl.ds(n0, EB)], ev_v)
            for k in range(8):
                kk = pl.ds(k * 16, 16)
                nv = n0 + k * 16 + lax.iota(jnp.int32, 16)
                comb_v[kk] = ev_v[kk] * (C * NPAD) + chunk * NPAD + nv
            pltpu.async_copy(h1f.at[comb_v], rows_v, sem).wait()
            pltpu.sync_copy(rows_v, h1sel.at[chunk].at[pl.ds(n0, EB)])

        plsc.subcore_barrier()

        @pl.when(sid == 0)
        def _writeback():
            pltpu.sync_copy(acc.at[pl.ds(0, N)], agg1.at[chunk])


def _sc_agg1(h1f, srcp, dstp, idxp, zbig):
    mesh = plsc.VectorSubcoreMesh(core_axis_name="c", subcore_axis_name="s")
    fn = pl.kernel(
        _sc_agg1_body, mesh=mesh,
        out_type=[
            jax.ShapeDtypeStruct((C, N, CW), jnp.float32),
            jax.ShapeDtypeStruct((C, NPAD, CW), jnp.float32),
        ],
        scratch_types=[
            pltpu.VMEM((EB,), jnp.int32),
            pltpu.VMEM((EB,), jnp.int32),
            pltpu.VMEM((EB,), jnp.int32),
            pltpu.VMEM((EB,), jnp.int32),
            pltpu.VMEM((EB, CW), jnp.float32),
            pltpu.SemaphoreType.DMA,
            pltpu.VMEM_SHARED((ACC_ROWS, CW), jnp.float32),
        ],
    )
    return fn(h1f, srcp, dstp, idxp, zbig)


def _comb_body(hs_ref, ag_ref, idx3_ref, wr_ref, wn_ref, b_ref, out_ref):
    hs = jnp.concatenate([hs_ref[c] for c in range(C)], axis=1)
    ag = jnp.concatenate([ag_ref[c] for c in range(C)], axis=1)
    idx = idx3_ref[0, 0, :]
    acc = jnp.zeros((BN, OUT_DIM), jnp.float32)
    for e in range(NEXP):
        m = (idx == e).astype(jnp.float32)[:, None]
        acc = acc + jnp.dot(hs * m, wr_ref[e], precision=DEF)
        acc = acc + jnp.dot(ag * m, wn_ref[e], precision=DEF)
    oh = (idx[:, None] == lax.broadcasted_iota(jnp.int32, (1, NEXP), 1)
          ).astype(jnp.float32)
    out_ref[...] = acc + jnp.dot(oh, b_ref[...], precision=HI)


def _combine(h1sel, agg1, idx3, We1_root, We1_nbr, be1):
    return pl.pallas_call(
        _comb_body,
        grid=(NBLK,),
        in_specs=[
            pl.BlockSpec((C, BN, CW), lambda i: (0, i, 0)),
            pl.BlockSpec((C, BN, CW), lambda i: (0, i, 0)),
            pl.BlockSpec((1, 1, BN), lambda i: (i, 0, 0)),
            pl.BlockSpec((NEXP, HIDDEN, OUT_DIM), lambda i: (0, 0, 0)),
            pl.BlockSpec((NEXP, HIDDEN, OUT_DIM), lambda i: (0, 0, 0)),
            pl.BlockSpec((NEXP, OUT_DIM), lambda i: (0, 0)),
        ],
        out_specs=pl.BlockSpec((BN, OUT_DIM), lambda i: (i, 0)),
        out_shape=jax.ShapeDtypeStruct((N, OUT_DIM), jnp.float32),
    )(h1sel, agg1, idx3, We1_root, We1_nbr, be1)


def kernel(x, edge_index, batch, W_enc, b_enc, W_r1, b_r1, W_r2, b_r2,
           We0_root, We0_nbr, be0, We1_root, We1_nbr, be1):
    src = edge_index[0].astype(jnp.int32)
    dst = edge_index[1].astype(jnp.int32)
    batch = batch.astype(jnp.int32)

    npad = EPAD - E
    srcp = jnp.concatenate([src, jnp.full((npad,), N + 8, jnp.int32)])
    dstp = jnp.concatenate([dst, jnp.full((npad,), TRASH, jnp.int32)])
    batchp = jnp.concatenate([batch, jnp.full((NPAD - N,), NGRAPH, jnp.int32)])
    zbig = jnp.zeros((ACC_ROWS, CW), jnp.float32)

    h, ht = _encoder(x, W_enc, b_enc)
    agg0 = _sc_agg0(ht, srcp, dstp, zbig)
    nc, ec = _counts(batchp.reshape(NPAD // 128, 128), src.reshape(E // 128, 128))
    idx3, hist = _router(h, batch.reshape(NBLK, 1, BN), nc, ec,
                         W_r1, b_r1, W_r2, b_r2)
    h1 = _layer0(h, agg0, We0_root, We0_nbr, be0)
    h1f = h1.reshape(NEXP * C * NPAD, CW)
    idxp = jnp.concatenate([idx3.reshape(N), jnp.zeros((NPAD - N,), jnp.int32)])
    agg1, h1sel = _sc_agg1(h1f, srcp, dstp, idxp, zbig)
    out = _combine(h1sel, agg1, idx3, We1_root, We1_nbr, be1)
    return out, hist.reshape(NEXP)
